# baseline (speedup 1.0000x reference)
import functools
import math

import jax
import jax.numpy as jnp
from jax import lax
from jax.experimental import pallas as pl
from jax.experimental.pallas import tpu as pltpu
from jax.experimental.pallas import tpu_sc as plsc

F32 = jnp.float32
NC = 2
NS = 16
CH = 40
NP = 50176


def _ln_rows(z, g, b):
    m = jnp.mean(z, axis=-1, keepdims=True)
    v = jnp.mean((z - m) * (z - m), axis=-1, keepdims=True)
    return (z - m) * lax.rsqrt(v + 1e-5) * g + b


def _full(shape):
    nd = len(shape)
    return pl.BlockSpec(shape, lambda i: (0,) * nd)


def _rows(r, w):
    return pl.BlockSpec((r, w), lambda i: (i, 0))


def _node_enc_body(pos_ref, x9_ref, Bm_ref, Ws_ref, Wc_ref, Wx_ref, b1_ref,
                   W2_ref, b2_ref, g_ref, bb_ref, h_ref):
    proj = (2.0 * math.pi) * jnp.dot(pos_ref[...], Bm_ref[...],
                                     preferred_element_type=F32)
    t = jnp.dot(jnp.sin(proj), Ws_ref[...], preferred_element_type=F32)
    t = t + jnp.dot(jnp.cos(proj), Wc_ref[...], preferred_element_type=F32)
    t = t + jnp.dot(x9_ref[...], Wx_ref[...], preferred_element_type=F32)
    t = jnp.maximum(t + b1_ref[...], 0.0)
    z = jnp.dot(t, W2_ref[...], preferred_element_type=F32) + b2_ref[...]
    h_ref[...] = _ln_rows(z, g_ref[...], bb_ref[...])


def _node_encoder(pos, x9, Bm, W1, b1, W2, b2, g, b, r=2000):
    n = pos.shape[0]
    Ws, Wc, Wx = W1[0:64], W1[64:128], W1[128:]
    return pl.pallas_call(
        _node_enc_body,
        grid=(n // r,),
        in_specs=[_rows(r, 2), _rows(r, x9.shape[1]), _full(Bm.shape),
                  _full(Ws.shape), _full(Wc.shape), _full(Wx.shape),
                  _full((1, 128)), _full(W2.shape), _full((1, 128)),
                  _full((1, 128)), _full((1, 128))],
        out_specs=_rows(r, 128),
        out_shape=jax.ShapeDtypeStruct((n, 128), F32),
    )(pos, x9, Bm, Ws, Wc, Wx, b1.reshape(1, -1), W2, b2.reshape(1, -1),
      g.reshape(1, -1), b.reshape(1, -1))


def _edge_enc_body(ea_ref, W1_ref, b1_ref, W2_ref, b2_ref, g_ref, bb_ref,
                   out_ref):
    t = jnp.dot(ea_ref[...], W1_ref[...], preferred_element_type=F32)
    t = jnp.maximum(t + b1_ref[...], 0.0)
    z = jnp.dot(t, W2_ref[...], preferred_element_type=F32) + b2_ref[...]
    out_ref[...] = _ln_rows(z, g_ref[...], bb_ref[...])


def _edge_encoder(ea, W1, b1, W2, b2, g, b, r=2000):
    n = ea.shape[0]
    return pl.pallas_call(
        _edge_enc_body,
        grid=(n // r,),
        in_specs=[_rows(r, ea.shape[1]), _full(W1.shape), _full((1, 128)),
                  _full(W2.shape), _full((1, 128)), _full((1, 128)),
                  _full((1, 128))],
        out_specs=_rows(r, 128),
        out_shape=jax.ShapeDtypeStruct((n, 128), F32),
    )(ea, W1, b1.reshape(1, -1), W2, b2.reshape(1, -1), g.reshape(1, -1),
      b.reshape(1, -1))


def _prep_body(h_ref, Ws_ref, Wd_ref, gs_ref, gd_ref):
    gs_ref[...] = jnp.dot(h_ref[...], Ws_ref[...], preferred_element_type=F32)
    gd_ref[...] = jnp.dot(h_ref[...], Wd_ref[...], preferred_element_type=F32)


def _prep_tables(h, Ws, Wd, r=2000):
    n = h.shape[0]
    return pl.pallas_call(
        _prep_body,
        grid=(n // r,),
        in_specs=[_rows(r, 128), _full(Ws.shape), _full(Wd.shape)],
        out_specs=[_rows(r, 128), _rows(r, 128)],
        out_shape=[jax.ShapeDtypeStruct((n, 128), F32),
                   jax.ShapeDtypeStruct((n, 128), F32)],
    )(h, Ws, Wd)


def _edge_mlp_body(e_ref, gath_ref, W1_ref, b1_ref, W2_ref, b2_ref, g_ref,
                   bb_ref, out_ref):
    t = jnp.dot(e_ref[...], W1_ref[...], preferred_element_type=F32)
    t = jnp.maximum(t + gath_ref[...] + b1_ref[...], 0.0)
    z = jnp.dot(t, W2_ref[...], preferred_element_type=F32) + b2_ref[...]
    out_ref[...] = e_ref[...] + _ln_rows(z, g_ref[...], bb_ref[...])


def _edge_mlp(e, gath, W1e, b1, W2, b2, g, b, r=2000):
    n = gath.shape[0]
    return pl.pallas_call(
        _edge_mlp_body,
        grid=(n // r,),
        in_specs=[_rows(r, 128), _rows(r, 128), _full(W1e.shape),
                  _full((1, 128)), _full(W2.shape), _full((1, 128)),
                  _full((1, 128)), _full((1, 128))],
        out_specs=_rows(r, 128),
        out_shape=jax.ShapeDtypeStruct((n, 128), F32),
    )(e, gath, W1e, b1.reshape(1, -1), W2, b2.reshape(1, -1),
      g.reshape(1, -1), b.reshape(1, -1))


def _node_mlp_body(h_ref, agg_ref, Wh_ref, Wa_ref, b1_ref, W2_ref, b2_ref,
                   g_ref, bb_ref, out_ref):
    t = jnp.dot(h_ref[...], Wh_ref[...], preferred_element_type=F32)
    a = jnp.concatenate([agg_ref[k] for k in range(4)], axis=-1)
    t = t + jnp.dot(a, Wa_ref[...], preferred_element_type=F32)
    t = jnp.maximum(t + b1_ref[...], 0.0)
    z = jnp.dot(t, W2_ref[...], preferred_element_type=F32) + b2_ref[...]
    out_ref[...] = h_ref[...] + _ln_rows(z, g_ref[...], bb_ref[...])


def _node_mlp(h, agg_all, Wh, Wa, b1, W2, b2, g, b, r=2000):
    n = h.shape[0]
    agg_spec = pl.BlockSpec((4, r, 32), lambda i: (0, i, 0))
    return pl.pallas_call(
        _node_mlp_body,
        grid=(n // r,),
        in_specs=[_rows(r, 128), agg_spec, _full(Wh.shape),
                  _full(Wa.shape),
                  _full((1, 128)), _full(W2.shape), _full((1, 128)),
                  _full((1, 128)), _full((1, 128))],
        out_specs=_rows(r, 128),
        out_shape=jax.ShapeDtypeStruct((n, 128), F32),
    )(h, agg_all, Wh, Wa, b1.reshape(1, -1), W2, b2.reshape(1, -1),
      g.reshape(1, -1), b.reshape(1, -1))


def _dec_body(h_ref, W1_ref, b1_ref, W2_ref, b2_ref, y_ref):
    t = jnp.dot(h_ref[...], W1_ref[...], preferred_element_type=F32)
    t = jnp.maximum(t + b1_ref[...], 0.0)
    y_ref[...] = jnp.dot(t, W2_ref[...],
                         preferred_element_type=F32) + b2_ref[...]


def _decoder(h, W1, b1, W2, b2, r=2000):
    n = h.shape[0]
    nd = W2.shape[1]
    return pl.pallas_call(
        _dec_body,
        grid=(n // r,),
        in_specs=[_rows(r, 128), _full(W1.shape), _full((1, 128)),
                  _full(W2.shape), _full((1, nd))],
        out_specs=_rows(r, nd),
        out_shape=jax.ShapeDtypeStruct((n, nd), F32),
    )(h, W1, b1.reshape(1, -1), W2, b2.reshape(1, -1))


@functools.lru_cache(maxsize=None)
def _make_gather_add(E, Hd):
    nw = NC * NS
    n_chunks = (E // nw) // CH
    mesh = plsc.VectorSubcoreMesh(core_axis_name="c", subcore_axis_name="s")

    K = 5

    @functools.partial(
        pl.kernel,
        out_type=jax.ShapeDtypeStruct((E, Hd), F32),
        mesh=mesh,
        scratch_types=[
            pltpu.VMEM((n_chunks, CH), jnp.int32),
            pltpu.VMEM((n_chunks, CH), jnp.int32),
            pltpu.VMEM((K, CH, Hd), F32),
            pltpu.SemaphoreType.DMA,
        ],
        compiler_params=pltpu.CompilerParams(use_tc_tiling_on_sc=False),
    )
    def gather_add(gs_hbm, gd_hbm, src_hbm, dst_hbm, out_hbm, src_v, dst_v,
                   rows_v, sem):
        wid = lax.axis_index("c") * NS + lax.axis_index("s")
        base = wid * n_chunks
        pltpu.sync_copy(src_hbm.at[wid], src_v)
        pltpu.sync_copy(dst_hbm.at[wid], dst_v)

        def group(g, carry):
            j0 = g * K
            ds = [pltpu.async_copy(gs_hbm.at[src_v.at[j0 + b]], rows_v.at[b],
                                   sem) for b in range(K)]
            for d in ds:
                d.wait()
            ds = [pltpu.async_copy(gd_hbm.at[dst_v.at[j0 + b]], rows_v.at[b],
                                   sem, add=True) for b in range(K)]
            for d in ds:
                d.wait()
            ds = [pltpu.async_copy(
                rows_v.at[b], out_hbm.at[pl.ds((base + j0 + b) * CH, CH)],
                sem) for b in range(K)]
            for d in ds:
                d.wait()
            return carry

        lax.fori_loop(0, n_chunks // K, group, 0)

    return gather_add


@functools.lru_cache(maxsize=None)
def _make_segment_sum(E, Hd):
    ec = Hd // 4
    n_chunks = (E // NS) // CH
    stripe = NP // NS
    mesh = plsc.VectorSubcoreMesh(core_axis_name="c", subcore_axis_name="s")

    K = 4

    @functools.partial(
        pl.kernel,
        out_type=jax.ShapeDtypeStruct((4, NP, ec), F32),
        mesh=mesh,
        scratch_types=[
            pltpu.VMEM((K, CH), jnp.int32),
            pltpu.VMEM((K, CH, Hd), F32),
            pltpu.VMEM((K, CH, ec), F32),
            pltpu.VMEM_SHARED((NP, ec), F32),
            pltpu.SemaphoreType.DMA,
        ],
        compiler_params=pltpu.CompilerParams(use_tc_tiling_on_sc=False),
    )
    def seg_sum(e_hbm, dst_hbm, zeros_hbm, agg_hbm, dst_v, rows_v, rows_c,
                slab, sem):
        core = lax.axis_index("c")
        sid = lax.axis_index("s")
        base = sid * n_chunks
        for p in range(2):
            pltpu.sync_copy(zeros_hbm, slab.at[pl.ds(sid * stripe, stripe)])
            plsc.subcore_barrier()

            def do_chunks(j0, nk, p=p):
                ds = [pltpu.async_copy(dst_hbm.at[sid, pl.ds(j0, nk)],
                                       dst_v.at[pl.ds(0, nk)], sem)]
                ds += [pltpu.async_copy(
                    e_hbm.at[pl.ds((base + j0 + b) * CH, CH)], rows_v.at[b],
                    sem) for b in range(nk)]
                for d in ds:
                    d.wait()
                for cv in range(NC):
                    col = ec * 2 * cv + ec * p

                    @pl.when(core == cv)
                    def _(col=col, nk=nk):
                        def cp(i, carry):
                            for b in range(nk):
                                rows_c[b, i, pl.ds(0, 16)] = (
                                    rows_v[b, i, pl.ds(col, 16)])
                                rows_c[b, i, pl.ds(16, 16)] = (
                                    rows_v[b, i, pl.ds(col + 16, 16)])
                            return carry

                        lax.fori_loop(0, CH, cp, 0)
                ds = [pltpu.async_copy(rows_c.at[b],
                                       slab.at[dst_v.at[b]],
                                       sem, add=True) for b in range(nk)]
                for d in ds:
                    d.wait()

            def group(g, carry):
                do_chunks(g * K, K)
                return carry

            lax.fori_loop(0, n_chunks // K, group, 0)
            do_chunks((n_chunks // K) * K, n_chunks % K)
            plsc.subcore_barrier()
            pltpu.sync_copy(
                slab.at[pl.ds(sid * stripe, stripe)],
                agg_hbm.at[2 * core + p, pl.ds(sid * stripe, stripe)])

    return seg_sum


def kernel(pos, x, edge_index, edge_attr, params):
    n = pos.shape[0]
    e_cnt = edge_attr.shape[0]
    nw = NC * NS
    src3 = edge_index[0].astype(jnp.int32).reshape(nw, (e_cnt // nw) // CH, CH)
    dst3 = edge_index[1].astype(jnp.int32).reshape(nw, (e_cnt // nw) // CH, CH)
    dst16 = edge_index[1].astype(jnp.int32).reshape(NS, (e_cnt // NS) // CH, CH)
    zeros = jnp.zeros((NP // NS, 32), F32)

    (We1, be1), (We2, be2) = params['node_enc']
    h = _node_encoder(pos, x[:, 2:], params['B'], We1, be1, We2, be2,
                      *params['node_enc_ln'])
    (Wf1, bf1), (Wf2, bf2) = params['edge_enc']
    e = _edge_encoder(edge_attr, Wf1, bf1, Wf2, bf2, *params['edge_enc_ln'])

    gather_add = _make_gather_add(e_cnt, 128)
    seg_sum = _make_segment_sum(e_cnt, 128)

    for blk in params['blocks']:
        (Wm1, bm1), (Wm2, bm2) = blk['edge_mlp']
        gs, gd = _prep_tables(h, Wm1[128:256], Wm1[256:384])
        gath = gather_add(gs, gd, src3, dst3)
        e = _edge_mlp(e, gath, Wm1[0:128], bm1, Wm2, bm2, *blk['edge_ln'])
        agg = seg_sum(e, dst16, zeros)
        (Wn1, bn1), (Wn2, bn2) = blk['node_mlp']
        h = _node_mlp(h, agg, Wn1[0:128], Wn1[128:256], bn1, Wn2, bn2,
                      *blk['node_ln'])

    (Wd1, bd1), (Wd2, bd2) = params['dec']
    return _decoder(h, Wd1, bd1, Wd2, bd2)


# speedup vs baseline: 1.5362x; 1.5362x over previous
import functools
import math

import jax
import jax.numpy as jnp
from jax import lax
from jax.experimental import pallas as pl
from jax.experimental.pallas import tpu as pltpu
from jax.experimental.pallas import tpu_sc as plsc

F32 = jnp.float32
NC = 2
NS = 16
CH = 40
NP = 50176


def _ln_rows(z, g, b):
    m = jnp.mean(z, axis=-1, keepdims=True)
    v = jnp.mean((z - m) * (z - m), axis=-1, keepdims=True)
    return (z - m) * lax.rsqrt(v + 1e-5) * g + b


def _full(shape):
    nd = len(shape)
    return pl.BlockSpec(shape, lambda i: (0,) * nd)


def _rows(r, w):
    return pl.BlockSpec((r, w), lambda i: (i, 0))


def _node_enc_body(pos_ref, x9_ref, Bm_ref, Ws_ref, Wc_ref, Wx_ref, b1_ref,
                   W2_ref, b2_ref, g_ref, bb_ref, h_ref):
    proj = (2.0 * math.pi) * jnp.dot(pos_ref[...], Bm_ref[...],
                                     preferred_element_type=F32)
    t = jnp.dot(jnp.sin(proj), Ws_ref[...], preferred_element_type=F32)
    t = t + jnp.dot(jnp.cos(proj), Wc_ref[...], preferred_element_type=F32)
    t = t + jnp.dot(x9_ref[...], Wx_ref[...], preferred_element_type=F32)
    t = jnp.maximum(t + b1_ref[...], 0.0)
    z = jnp.dot(t, W2_ref[...], preferred_element_type=F32) + b2_ref[...]
    h_ref[...] = _ln_rows(z, g_ref[...], bb_ref[...])


def _node_encoder(pos, x9, Bm, W1, b1, W2, b2, g, b, r=2000):
    n = pos.shape[0]
    Ws, Wc, Wx = W1[0:64], W1[64:128], W1[128:]
    return pl.pallas_call(
        _node_enc_body,
        grid=(n // r,),
        in_specs=[_rows(r, 2), _rows(r, x9.shape[1]), _full(Bm.shape),
                  _full(Ws.shape), _full(Wc.shape), _full(Wx.shape),
                  _full((1, 128)), _full(W2.shape), _full((1, 128)),
                  _full((1, 128)), _full((1, 128))],
        out_specs=_rows(r, 128),
        out_shape=jax.ShapeDtypeStruct((n, 128), F32),
    )(pos, x9, Bm, Ws, Wc, Wx, b1.reshape(1, -1), W2, b2.reshape(1, -1),
      g.reshape(1, -1), b.reshape(1, -1))


def _edge_enc_body(ea_ref, W1_ref, b1_ref, W2_ref, b2_ref, g_ref, bb_ref,
                   out_ref):
    t = jnp.dot(ea_ref[...], W1_ref[...], preferred_element_type=F32)
    t = jnp.maximum(t + b1_ref[...], 0.0)
    z = jnp.dot(t, W2_ref[...], preferred_element_type=F32) + b2_ref[...]
    out_ref[...] = _ln_rows(z, g_ref[...], bb_ref[...])


def _edge_encoder(ea, W1, b1, W2, b2, g, b, r=2000):
    n = ea.shape[0]
    return pl.pallas_call(
        _edge_enc_body,
        grid=(n // r,),
        in_specs=[_rows(r, ea.shape[1]), _full(W1.shape), _full((1, 128)),
                  _full(W2.shape), _full((1, 128)), _full((1, 128)),
                  _full((1, 128))],
        out_specs=_rows(r, 128),
        out_shape=jax.ShapeDtypeStruct((n, 128), F32),
    )(ea, W1, b1.reshape(1, -1), W2, b2.reshape(1, -1), g.reshape(1, -1),
      b.reshape(1, -1))


def _prep_body(h_ref, Ws_ref, Wd_ref, gs_ref, gd_ref):
    gs_ref[...] = jnp.dot(h_ref[...], Ws_ref[...], preferred_element_type=F32)
    gd_ref[...] = jnp.dot(h_ref[...], Wd_ref[...], preferred_element_type=F32)


def _prep_tables(h, Ws, Wd, r=2000):
    n = h.shape[0]
    return pl.pallas_call(
        _prep_body,
        grid=(n // r,),
        in_specs=[_rows(r, 128), _full(Ws.shape), _full(Wd.shape)],
        out_specs=[_rows(r, 128), _rows(r, 128)],
        out_shape=[jax.ShapeDtypeStruct((n, 128), F32),
                   jax.ShapeDtypeStruct((n, 128), F32)],
    )(h, Ws, Wd)


def _edge_mlp_body(e_ref, gath_ref, W1_ref, b1_ref, W2_ref, b2_ref, g_ref,
                   bb_ref, out_ref):
    t = jnp.dot(e_ref[...], W1_ref[...], preferred_element_type=F32)
    t = jnp.maximum(t + gath_ref[...] + b1_ref[...], 0.0)
    z = jnp.dot(t, W2_ref[...], preferred_element_type=F32) + b2_ref[...]
    out_ref[...] = e_ref[...] + _ln_rows(z, g_ref[...], bb_ref[...])


def _edge_mlp(e, gath, W1e, b1, W2, b2, g, b, r=2000):
    n = gath.shape[0]
    return pl.pallas_call(
        _edge_mlp_body,
        grid=(n // r,),
        in_specs=[_rows(r, 128), _rows(r, 128), _full(W1e.shape),
                  _full((1, 128)), _full(W2.shape), _full((1, 128)),
                  _full((1, 128)), _full((1, 128))],
        out_specs=_rows(r, 128),
        out_shape=jax.ShapeDtypeStruct((n, 128), F32),
    )(e, gath, W1e, b1.reshape(1, -1), W2, b2.reshape(1, -1),
      g.reshape(1, -1), b.reshape(1, -1))


def _node_mlp_body(h_ref, agg_ref, Wh_ref, Wa_ref, b1_ref, W2_ref, b2_ref,
                   g_ref, bb_ref, out_ref):
    t = jnp.dot(h_ref[...], Wh_ref[...], preferred_element_type=F32)
    a = jnp.concatenate([agg_ref[k] for k in range(4)], axis=-1)
    t = t + jnp.dot(a, Wa_ref[...], preferred_element_type=F32)
    t = jnp.maximum(t + b1_ref[...], 0.0)
    z = jnp.dot(t, W2_ref[...], preferred_element_type=F32) + b2_ref[...]
    out_ref[...] = h_ref[...] + _ln_rows(z, g_ref[...], bb_ref[...])


def _node_mlp(h, agg_all, Wh, Wa, b1, W2, b2, g, b, r=2000):
    n = h.shape[0]
    agg_spec = pl.BlockSpec((4, r, 32), lambda i: (0, i, 0))
    return pl.pallas_call(
        _node_mlp_body,
        grid=(n // r,),
        in_specs=[_rows(r, 128), agg_spec, _full(Wh.shape),
                  _full(Wa.shape),
                  _full((1, 128)), _full(W2.shape), _full((1, 128)),
                  _full((1, 128)), _full((1, 128))],
        out_specs=_rows(r, 128),
        out_shape=jax.ShapeDtypeStruct((n, 128), F32),
    )(h, agg_all, Wh, Wa, b1.reshape(1, -1), W2, b2.reshape(1, -1),
      g.reshape(1, -1), b.reshape(1, -1))


def _dec_body(h_ref, W1_ref, b1_ref, W2_ref, b2_ref, y_ref):
    t = jnp.dot(h_ref[...], W1_ref[...], preferred_element_type=F32)
    t = jnp.maximum(t + b1_ref[...], 0.0)
    y_ref[...] = jnp.dot(t, W2_ref[...],
                         preferred_element_type=F32) + b2_ref[...]


def _decoder(h, W1, b1, W2, b2, r=2000):
    n = h.shape[0]
    nd = W2.shape[1]
    return pl.pallas_call(
        _dec_body,
        grid=(n // r,),
        in_specs=[_rows(r, 128), _full(W1.shape), _full((1, 128)),
                  _full(W2.shape), _full((1, nd))],
        out_specs=_rows(r, nd),
        out_shape=jax.ShapeDtypeStruct((n, nd), F32),
    )(h, W1, b1.reshape(1, -1), W2, b2.reshape(1, -1))


@functools.lru_cache(maxsize=None)
def _make_gather_add(E, Hd):
    nw = NC * NS
    n_chunks = (E // nw) // CH
    mesh = plsc.VectorSubcoreMesh(core_axis_name="c", subcore_axis_name="s")

    K = 5

    @functools.partial(
        pl.kernel,
        out_type=jax.ShapeDtypeStruct((E, Hd), F32),
        mesh=mesh,
        scratch_types=[
            pltpu.VMEM((n_chunks, CH), jnp.int32),
            pltpu.VMEM((n_chunks, CH), jnp.int32),
            pltpu.VMEM((K, CH, Hd), F32),
            pltpu.SemaphoreType.DMA,
        ],
        compiler_params=pltpu.CompilerParams(use_tc_tiling_on_sc=False),
    )
    def gather_add(gs_hbm, gd_hbm, src_hbm, dst_hbm, out_hbm, src_v, dst_v,
                   rows_v, sem):
        wid = lax.axis_index("c") * NS + lax.axis_index("s")
        base = wid * n_chunks
        pltpu.sync_copy(src_hbm.at[wid], src_v)
        pltpu.sync_copy(dst_hbm.at[wid], dst_v)

        def group(g, carry):
            j0 = g * K
            ds = [pltpu.async_copy(gs_hbm.at[src_v.at[j0 + b]], rows_v.at[b],
                                   sem) for b in range(K)]
            for d in ds:
                d.wait()
            ds = [pltpu.async_copy(gd_hbm.at[dst_v.at[j0 + b]], rows_v.at[b],
                                   sem, add=True) for b in range(K)]
            for d in ds:
                d.wait()
            ds = [pltpu.async_copy(
                rows_v.at[b], out_hbm.at[pl.ds((base + j0 + b) * CH, CH)],
                sem) for b in range(K)]
            for d in ds:
                d.wait()
            return carry

        lax.fori_loop(0, n_chunks // K, group, 0)

    return gather_add


@functools.lru_cache(maxsize=None)
def _make_segment_sum(E, Hd):
    ec = Hd // 4
    n_chunks = (E // NS) // CH
    stripe = NP // NS
    mesh = plsc.VectorSubcoreMesh(core_axis_name="c", subcore_axis_name="s")

    K = 5

    @functools.partial(
        pl.kernel,
        out_type=jax.ShapeDtypeStruct((4, NP, ec), F32),
        mesh=mesh,
        scratch_types=[
            pltpu.VMEM((K, CH), jnp.int32),
            pltpu.VMEM((K, CH, ec), F32),
            pltpu.VMEM_SHARED((NP, ec), F32),
            pltpu.SemaphoreType.DMA,
        ],
        compiler_params=pltpu.CompilerParams(use_tc_tiling_on_sc=False),
    )
    def seg_sum(e_hbm, dst_hbm, zeros_hbm, agg_hbm, dst_v, rows_c, slab, sem):
        core = lax.axis_index("c")
        sid = lax.axis_index("s")
        base = sid * n_chunks
        for p in range(2):
            pltpu.sync_copy(zeros_hbm, slab.at[pl.ds(sid * stripe, stripe)])
            plsc.subcore_barrier()
            col = ec * 2 * core + ec * p

            def group(g, carry, col=col, p=p):
                j0 = g * K
                ds = [pltpu.async_copy(dst_hbm.at[sid, pl.ds(j0, K)],
                                       dst_v, sem)]
                ds += [pltpu.async_copy(
                    e_hbm.at[pl.ds((base + j0 + b) * CH, CH),
                             pl.ds(col, ec)], rows_c.at[b],
                    sem) for b in range(K)]
                for d in ds:
                    d.wait()
                ds = [pltpu.async_copy(rows_c.at[b],
                                       slab.at[dst_v.at[b]],
                                       sem, add=True) for b in range(K)]
                for d in ds:
                    d.wait()
                return carry

            lax.fori_loop(0, n_chunks // K, group, 0)
            plsc.subcore_barrier()
            pltpu.sync_copy(
                slab.at[pl.ds(sid * stripe, stripe)],
                agg_hbm.at[2 * core + p, pl.ds(sid * stripe, stripe)])

    return seg_sum


def kernel(pos, x, edge_index, edge_attr, params):
    n = pos.shape[0]
    e_cnt = edge_attr.shape[0]
    nw = NC * NS
    src3 = edge_index[0].astype(jnp.int32).reshape(nw, (e_cnt // nw) // CH, CH)
    dst3 = edge_index[1].astype(jnp.int32).reshape(nw, (e_cnt // nw) // CH, CH)
    dst16 = edge_index[1].astype(jnp.int32).reshape(NS, (e_cnt // NS) // CH, CH)
    zeros = jnp.zeros((NP // NS, 32), F32)

    (We1, be1), (We2, be2) = params['node_enc']
    h = _node_encoder(pos, x[:, 2:], params['B'], We1, be1, We2, be2,
                      *params['node_enc_ln'])
    (Wf1, bf1), (Wf2, bf2) = params['edge_enc']
    e = _edge_encoder(edge_attr, Wf1, bf1, Wf2, bf2, *params['edge_enc_ln'])

    gather_add = _make_gather_add(e_cnt, 128)
    seg_sum = _make_segment_sum(e_cnt, 128)

    for blk in params['blocks']:
        (Wm1, bm1), (Wm2, bm2) = blk['edge_mlp']
        gs, gd = _prep_tables(h, Wm1[128:256], Wm1[256:384])
        gath = gather_add(gs, gd, src3, dst3)
        e = _edge_mlp(e, gath, Wm1[0:128], bm1, Wm2, bm2, *blk['edge_ln'])
        agg = seg_sum(e, dst16, zeros)
        (Wn1, bn1), (Wn2, bn2) = blk['node_mlp']
        h = _node_mlp(h, agg, Wn1[0:128], Wn1[128:256], bn1, Wn2, bn2,
                      *blk['node_ln'])

    (Wd1, bd1), (Wd2, bd2) = params['dec']
    return _decoder(h, Wd1, bd1, Wd2, bd2)


# speedup vs baseline: 1.5842x; 1.0313x over previous
import functools
import math

import jax
import jax.numpy as jnp
from jax import lax
from jax.experimental import pallas as pl
from jax.experimental.pallas import tpu as pltpu
from jax.experimental.pallas import tpu_sc as plsc

F32 = jnp.float32
NC = 2
NS = 16
CH = 40
NP = 50176


def _ln_rows(z, g, b):
    m = jnp.mean(z, axis=-1, keepdims=True)
    v = jnp.mean((z - m) * (z - m), axis=-1, keepdims=True)
    return (z - m) * lax.rsqrt(v + 1e-5) * g + b


def _full(shape):
    nd = len(shape)
    return pl.BlockSpec(shape, lambda i: (0,) * nd)


def _rows(r, w):
    return pl.BlockSpec((r, w), lambda i: (i, 0))


def _node_enc_body(pos_ref, x9_ref, Bm_ref, Ws_ref, Wc_ref, Wx_ref, b1_ref,
                   W2_ref, b2_ref, g_ref, bb_ref, h_ref):
    proj = (2.0 * math.pi) * jnp.dot(pos_ref[...], Bm_ref[...],
                                     preferred_element_type=F32)
    t = jnp.dot(jnp.sin(proj), Ws_ref[...], preferred_element_type=F32)
    t = t + jnp.dot(jnp.cos(proj), Wc_ref[...], preferred_element_type=F32)
    t = t + jnp.dot(x9_ref[...], Wx_ref[...], preferred_element_type=F32)
    t = jnp.maximum(t + b1_ref[...], 0.0)
    z = jnp.dot(t, W2_ref[...], preferred_element_type=F32) + b2_ref[...]
    h_ref[...] = _ln_rows(z, g_ref[...], bb_ref[...])


def _node_encoder(pos, x9, Bm, W1, b1, W2, b2, g, b, r=2000):
    n = pos.shape[0]
    Ws, Wc, Wx = W1[0:64], W1[64:128], W1[128:]
    return pl.pallas_call(
        _node_enc_body,
        grid=(n // r,),
        in_specs=[_rows(r, 2), _rows(r, x9.shape[1]), _full(Bm.shape),
                  _full(Ws.shape), _full(Wc.shape), _full(Wx.shape),
                  _full((1, 128)), _full(W2.shape), _full((1, 128)),
                  _full((1, 128)), _full((1, 128))],
        out_specs=_rows(r, 128),
        out_shape=jax.ShapeDtypeStruct((n, 128), F32),
    )(pos, x9, Bm, Ws, Wc, Wx, b1.reshape(1, -1), W2, b2.reshape(1, -1),
      g.reshape(1, -1), b.reshape(1, -1))


def _edge_enc_body(ea_ref, W1_ref, b1_ref, W2_ref, b2_ref, g_ref, bb_ref,
                   out_ref):
    t = jnp.dot(ea_ref[...], W1_ref[...], preferred_element_type=F32)
    t = jnp.maximum(t + b1_ref[...], 0.0)
    z = jnp.dot(t, W2_ref[...], preferred_element_type=F32) + b2_ref[...]
    out_ref[...] = _ln_rows(z, g_ref[...], bb_ref[...])


def _edge_encoder(ea, W1, b1, W2, b2, g, b, r=2000):
    n = ea.shape[0]
    return pl.pallas_call(
        _edge_enc_body,
        grid=(n // r,),
        in_specs=[_rows(r, ea.shape[1]), _full(W1.shape), _full((1, 128)),
                  _full(W2.shape), _full((1, 128)), _full((1, 128)),
                  _full((1, 128))],
        out_specs=_rows(r, 128),
        out_shape=jax.ShapeDtypeStruct((n, 128), F32),
    )(ea, W1, b1.reshape(1, -1), W2, b2.reshape(1, -1), g.reshape(1, -1),
      b.reshape(1, -1))


def _prep_body(h_ref, Ws_ref, Wd_ref, gs_ref, gd_ref):
    gs_ref[...] = jnp.dot(h_ref[...], Ws_ref[...], preferred_element_type=F32)
    gd_ref[...] = jnp.dot(h_ref[...], Wd_ref[...], preferred_element_type=F32)


def _prep_tables(h, Ws, Wd, r=2000):
    n = h.shape[0]
    return pl.pallas_call(
        _prep_body,
        grid=(n // r,),
        in_specs=[_rows(r, 128), _full(Ws.shape), _full(Wd.shape)],
        out_specs=[_rows(r, 128), _rows(r, 128)],
        out_shape=[jax.ShapeDtypeStruct((n, 128), F32),
                   jax.ShapeDtypeStruct((n, 128), F32)],
    )(h, Ws, Wd)


def _edge_mlp_body(e_ref, gath_ref, W1_ref, b1_ref, W2_ref, b2_ref, g_ref,
                   bb_ref, out_ref):
    t = jnp.dot(e_ref[...], W1_ref[...], preferred_element_type=F32)
    t = jnp.maximum(t + gath_ref[...] + b1_ref[...], 0.0)
    z = jnp.dot(t, W2_ref[...], preferred_element_type=F32) + b2_ref[...]
    out_ref[...] = e_ref[...] + _ln_rows(z, g_ref[...], bb_ref[...])


def _edge_mlp(e, gath, W1e, b1, W2, b2, g, b, r=2000):
    n = gath.shape[0]
    return pl.pallas_call(
        _edge_mlp_body,
        grid=(n // r,),
        in_specs=[_rows(r, 128), _rows(r, 128), _full(W1e.shape),
                  _full((1, 128)), _full(W2.shape), _full((1, 128)),
                  _full((1, 128)), _full((1, 128))],
        out_specs=_rows(r, 128),
        out_shape=jax.ShapeDtypeStruct((n, 128), F32),
    )(e, gath, W1e, b1.reshape(1, -1), W2, b2.reshape(1, -1),
      g.reshape(1, -1), b.reshape(1, -1))


def _node_mlp_body(h_ref, agg_ref, Wh_ref, Wa_ref, b1_ref, W2_ref, b2_ref,
                   g_ref, bb_ref, out_ref):
    t = jnp.dot(h_ref[...], Wh_ref[...], preferred_element_type=F32)
    t = t + jnp.dot(agg_ref[...], Wa_ref[...], preferred_element_type=F32)
    t = jnp.maximum(t + b1_ref[...], 0.0)
    z = jnp.dot(t, W2_ref[...], preferred_element_type=F32) + b2_ref[...]
    out_ref[...] = h_ref[...] + _ln_rows(z, g_ref[...], bb_ref[...])


def _node_mlp(h, agg_all, Wh, Wa, b1, W2, b2, g, b, r=2000):
    n = h.shape[0]
    return pl.pallas_call(
        _node_mlp_body,
        grid=(n // r,),
        in_specs=[_rows(r, 128), _rows(r, 128), _full(Wh.shape),
                  _full(Wa.shape),
                  _full((1, 128)), _full(W2.shape), _full((1, 128)),
                  _full((1, 128)), _full((1, 128))],
        out_specs=_rows(r, 128),
        out_shape=jax.ShapeDtypeStruct((n, 128), F32),
    )(h, agg_all, Wh, Wa, b1.reshape(1, -1), W2, b2.reshape(1, -1),
      g.reshape(1, -1), b.reshape(1, -1))


def _dec_body(h_ref, W1_ref, b1_ref, W2_ref, b2_ref, y_ref):
    t = jnp.dot(h_ref[...], W1_ref[...], preferred_element_type=F32)
    t = jnp.maximum(t + b1_ref[...], 0.0)
    y_ref[...] = jnp.dot(t, W2_ref[...],
                         preferred_element_type=F32) + b2_ref[...]


def _decoder(h, W1, b1, W2, b2, r=2000):
    n = h.shape[0]
    nd = W2.shape[1]
    return pl.pallas_call(
        _dec_body,
        grid=(n // r,),
        in_specs=[_rows(r, 128), _full(W1.shape), _full((1, 128)),
                  _full(W2.shape), _full((1, nd))],
        out_specs=_rows(r, nd),
        out_shape=jax.ShapeDtypeStruct((n, nd), F32),
    )(h, W1, b1.reshape(1, -1), W2, b2.reshape(1, -1))


@functools.lru_cache(maxsize=None)
def _make_gather_add(E, Hd):
    nw = NC * NS
    n_chunks = (E // nw) // CH
    mesh = plsc.VectorSubcoreMesh(core_axis_name="c", subcore_axis_name="s")

    K = 5

    @functools.partial(
        pl.kernel,
        out_type=jax.ShapeDtypeStruct((E, Hd), F32),
        mesh=mesh,
        scratch_types=[
            pltpu.VMEM((n_chunks, CH), jnp.int32),
            pltpu.VMEM((n_chunks, CH), jnp.int32),
            pltpu.VMEM((K, CH, Hd), F32),
            pltpu.SemaphoreType.DMA,
        ],
        compiler_params=pltpu.CompilerParams(use_tc_tiling_on_sc=False),
    )
    def gather_add(gs_hbm, gd_hbm, src_hbm, dst_hbm, out_hbm, src_v, dst_v,
                   rows_v, sem):
        wid = lax.axis_index("c") * NS + lax.axis_index("s")
        base = wid * n_chunks
        pltpu.sync_copy(src_hbm.at[wid], src_v)
        pltpu.sync_copy(dst_hbm.at[wid], dst_v)

        def group(g, carry):
            j0 = g * K
            ds = [pltpu.async_copy(gs_hbm.at[src_v.at[j0 + b]], rows_v.at[b],
                                   sem) for b in range(K)]
            for d in ds:
                d.wait()
            ds = [pltpu.async_copy(gd_hbm.at[dst_v.at[j0 + b]], rows_v.at[b],
                                   sem, add=True) for b in range(K)]
            for d in ds:
                d.wait()
            ds = [pltpu.async_copy(
                rows_v.at[b], out_hbm.at[pl.ds((base + j0 + b) * CH, CH)],
                sem) for b in range(K)]
            for d in ds:
                d.wait()
            return carry

        lax.fori_loop(0, n_chunks // K, group, 0)

    return gather_add


@functools.lru_cache(maxsize=None)
def _make_segment_sum(E, Hd):
    ec = Hd // 4
    n_chunks = (E // NS) // CH
    stripe = NP // NS
    mesh = plsc.VectorSubcoreMesh(core_axis_name="c", subcore_axis_name="s")

    K = 5

    @functools.partial(
        pl.kernel,
        out_type=jax.ShapeDtypeStruct((NP, Hd), F32),
        mesh=mesh,
        scratch_types=[
            pltpu.VMEM((K, CH), jnp.int32),
            pltpu.VMEM((K, CH, ec), F32),
            pltpu.VMEM_SHARED((NP, ec), F32),
            pltpu.SemaphoreType.DMA,
        ],
        compiler_params=pltpu.CompilerParams(use_tc_tiling_on_sc=False),
    )
    def seg_sum(e_hbm, dst_hbm, zeros_hbm, agg_hbm, dst_v, rows_c, slab, sem):
        core = lax.axis_index("c")
        sid = lax.axis_index("s")
        base = sid * n_chunks
        for p in range(2):
            pltpu.sync_copy(zeros_hbm, slab.at[pl.ds(sid * stripe, stripe)])
            plsc.subcore_barrier()
            col = ec * 2 * core + ec * p

            def group(g, carry, col=col, p=p):
                j0 = g * K
                ds = [pltpu.async_copy(dst_hbm.at[sid, pl.ds(j0, K)],
                                       dst_v, sem)]
                ds += [pltpu.async_copy(
                    e_hbm.at[pl.ds((base + j0 + b) * CH, CH),
                             pl.ds(col, ec)], rows_c.at[b],
                    sem) for b in range(K)]
                for d in ds:
                    d.wait()
                ds = [pltpu.async_copy(rows_c.at[b],
                                       slab.at[dst_v.at[b]],
                                       sem, add=True) for b in range(K)]
                for d in ds:
                    d.wait()
                return carry

            lax.fori_loop(0, n_chunks // K, group, 0)
            plsc.subcore_barrier()
            pltpu.sync_copy(
                slab.at[pl.ds(sid * stripe, stripe)],
                agg_hbm.at[pl.ds(sid * stripe, stripe), pl.ds(col, ec)])

    return seg_sum


def kernel(pos, x, edge_index, edge_attr, params):
    n = pos.shape[0]
    e_cnt = edge_attr.shape[0]
    nw = NC * NS
    src3 = edge_index[0].astype(jnp.int32).reshape(nw, (e_cnt // nw) // CH, CH)
    dst3 = edge_index[1].astype(jnp.int32).reshape(nw, (e_cnt // nw) // CH, CH)
    dst16 = edge_index[1].astype(jnp.int32).reshape(NS, (e_cnt // NS) // CH, CH)
    zeros = jnp.zeros((NP // NS, 32), F32)

    (We1, be1), (We2, be2) = params['node_enc']
    h = _node_encoder(pos, x[:, 2:], params['B'], We1, be1, We2, be2,
                      *params['node_enc_ln'])
    (Wf1, bf1), (Wf2, bf2) = params['edge_enc']
    e = _edge_encoder(edge_attr, Wf1, bf1, Wf2, bf2, *params['edge_enc_ln'])

    gather_add = _make_gather_add(e_cnt, 128)
    seg_sum = _make_segment_sum(e_cnt, 128)

    for blk in params['blocks']:
        (Wm1, bm1), (Wm2, bm2) = blk['edge_mlp']
        gs, gd = _prep_tables(h, Wm1[128:256], Wm1[256:384])
        gath = gather_add(gs, gd, src3, dst3)
        e = _edge_mlp(e, gath, Wm1[0:128], bm1, Wm2, bm2, *blk['edge_ln'])
        agg = seg_sum(e, dst16, zeros)
        (Wn1, bn1), (Wn2, bn2) = blk['node_mlp']
        h = _node_mlp(h, agg, Wn1[0:128], Wn1[128:256], bn1, Wn2, bn2,
                      *blk['node_ln'])

    (Wd1, bd1), (Wd2, bd2) = params['dec']
    return _decoder(h, Wd1, bd1, Wd2, bd2)


# speedup vs baseline: 1.8619x; 1.1753x over previous
import functools
import math

import jax
import jax.numpy as jnp
from jax import lax
from jax.experimental import pallas as pl
from jax.experimental.pallas import tpu as pltpu
from jax.experimental.pallas import tpu_sc as plsc

F32 = jnp.float32
NC = 2
NS = 16
CH = 40
NP = 50176


def _ln_rows(z, g, b):
    m = jnp.mean(z, axis=-1, keepdims=True)
    v = jnp.mean((z - m) * (z - m), axis=-1, keepdims=True)
    return (z - m) * lax.rsqrt(v + 1e-5) * g + b


def _full(shape):
    nd = len(shape)
    return pl.BlockSpec(shape, lambda i: (0,) * nd)


def _rows(r, w):
    return pl.BlockSpec((r, w), lambda i: (i, 0))


def _node_enc_body(pos_ref, x9_ref, Bm_ref, Ws_ref, Wc_ref, Wx_ref, b1_ref,
                   W2_ref, b2_ref, g_ref, bb_ref, h_ref):
    proj = (2.0 * math.pi) * jnp.dot(pos_ref[...], Bm_ref[...],
                                     preferred_element_type=F32)
    t = jnp.dot(jnp.sin(proj), Ws_ref[...], preferred_element_type=F32)
    t = t + jnp.dot(jnp.cos(proj), Wc_ref[...], preferred_element_type=F32)
    t = t + jnp.dot(x9_ref[...], Wx_ref[...], preferred_element_type=F32)
    t = jnp.maximum(t + b1_ref[...], 0.0)
    z = jnp.dot(t, W2_ref[...], preferred_element_type=F32) + b2_ref[...]
    h_ref[...] = _ln_rows(z, g_ref[...], bb_ref[...])


def _node_encoder(pos, x9, Bm, W1, b1, W2, b2, g, b, r=2000):
    n = pos.shape[0]
    Ws, Wc, Wx = W1[0:64], W1[64:128], W1[128:]
    return pl.pallas_call(
        _node_enc_body,
        grid=(n // r,),
        in_specs=[_rows(r, 2), _rows(r, x9.shape[1]), _full(Bm.shape),
                  _full(Ws.shape), _full(Wc.shape), _full(Wx.shape),
                  _full((1, 128)), _full(W2.shape), _full((1, 128)),
                  _full((1, 128)), _full((1, 128))],
        out_specs=_rows(r, 128),
        out_shape=jax.ShapeDtypeStruct((n, 128), F32),
    )(pos, x9, Bm, Ws, Wc, Wx, b1.reshape(1, -1), W2, b2.reshape(1, -1),
      g.reshape(1, -1), b.reshape(1, -1))


def _edge_enc_body(ea_ref, W1_ref, b1_ref, W2_ref, b2_ref, g_ref, bb_ref,
                   out_ref):
    t = jnp.dot(ea_ref[...], W1_ref[...], preferred_element_type=F32)
    t = jnp.maximum(t + b1_ref[...], 0.0)
    z = jnp.dot(t, W2_ref[...], preferred_element_type=F32) + b2_ref[...]
    out_ref[...] = _ln_rows(z, g_ref[...], bb_ref[...])


def _edge_encoder(ea, W1, b1, W2, b2, g, b, r=2000):
    n = ea.shape[0]
    return pl.pallas_call(
        _edge_enc_body,
        grid=(n // r,),
        in_specs=[_rows(r, ea.shape[1]), _full(W1.shape), _full((1, 128)),
                  _full(W2.shape), _full((1, 128)), _full((1, 128)),
                  _full((1, 128))],
        out_specs=_rows(r, 128),
        out_shape=jax.ShapeDtypeStruct((n, 128), F32),
    )(ea, W1, b1.reshape(1, -1), W2, b2.reshape(1, -1), g.reshape(1, -1),
      b.reshape(1, -1))


def _prep_body(h_ref, Ws_ref, Wd_ref, gs_ref, gd_ref):
    gs_ref[...] = jnp.dot(h_ref[...], Ws_ref[...], preferred_element_type=F32)
    gd_ref[...] = jnp.dot(h_ref[...], Wd_ref[...], preferred_element_type=F32)


def _prep_tables(h, Ws, Wd, r=2000):
    n = h.shape[0]
    return pl.pallas_call(
        _prep_body,
        grid=(n // r,),
        in_specs=[_rows(r, 128), _full(Ws.shape), _full(Wd.shape)],
        out_specs=[_rows(r, 128), _rows(r, 128)],
        out_shape=[jax.ShapeDtypeStruct((n, 128), F32),
                   jax.ShapeDtypeStruct((n, 128), F32)],
    )(h, Ws, Wd)


def _edge_mlp_body(e_ref, gath_ref, W1_ref, b1_ref, W2_ref, b2_ref, g_ref,
                   bb_ref, out_ref):
    t = jnp.dot(e_ref[...], W1_ref[...], preferred_element_type=F32)
    t = jnp.maximum(t + gath_ref[...] + b1_ref[...], 0.0)
    z = jnp.dot(t, W2_ref[...], preferred_element_type=F32) + b2_ref[...]
    out_ref[...] = e_ref[...] + _ln_rows(z, g_ref[...], bb_ref[...])


def _edge_mlp(e, gath, W1e, b1, W2, b2, g, b, r=2000):
    n = gath.shape[0]
    return pl.pallas_call(
        _edge_mlp_body,
        grid=(n // r,),
        in_specs=[_rows(r, 128), _rows(r, 128), _full(W1e.shape),
                  _full((1, 128)), _full(W2.shape), _full((1, 128)),
                  _full((1, 128)), _full((1, 128))],
        out_specs=_rows(r, 128),
        out_shape=jax.ShapeDtypeStruct((n, 128), F32),
    )(e, gath, W1e, b1.reshape(1, -1), W2, b2.reshape(1, -1),
      g.reshape(1, -1), b.reshape(1, -1))


def _node_mlp_body(h_ref, agg_ref, Wh_ref, Wa_ref, b1_ref, W2_ref, b2_ref,
                   g_ref, bb_ref, out_ref):
    t = jnp.dot(h_ref[...], Wh_ref[...], preferred_element_type=F32)
    t = t + jnp.dot(agg_ref[...], Wa_ref[...], preferred_element_type=F32)
    t = jnp.maximum(t + b1_ref[...], 0.0)
    z = jnp.dot(t, W2_ref[...], preferred_element_type=F32) + b2_ref[...]
    out_ref[...] = h_ref[...] + _ln_rows(z, g_ref[...], bb_ref[...])


def _node_mlp(h, agg_all, Wh, Wa, b1, W2, b2, g, b, r=2000):
    n = h.shape[0]
    return pl.pallas_call(
        _node_mlp_body,
        grid=(n // r,),
        in_specs=[_rows(r, 128), _rows(r, 128), _full(Wh.shape),
                  _full(Wa.shape),
                  _full((1, 128)), _full(W2.shape), _full((1, 128)),
                  _full((1, 128)), _full((1, 128))],
        out_specs=_rows(r, 128),
        out_shape=jax.ShapeDtypeStruct((n, 128), F32),
    )(h, agg_all, Wh, Wa, b1.reshape(1, -1), W2, b2.reshape(1, -1),
      g.reshape(1, -1), b.reshape(1, -1))


def _dec_body(h_ref, W1_ref, b1_ref, W2_ref, b2_ref, y_ref):
    t = jnp.dot(h_ref[...], W1_ref[...], preferred_element_type=F32)
    t = jnp.maximum(t + b1_ref[...], 0.0)
    y_ref[...] = jnp.dot(t, W2_ref[...],
                         preferred_element_type=F32) + b2_ref[...]


def _decoder(h, W1, b1, W2, b2, r=2000):
    n = h.shape[0]
    nd = W2.shape[1]
    return pl.pallas_call(
        _dec_body,
        grid=(n // r,),
        in_specs=[_rows(r, 128), _full(W1.shape), _full((1, 128)),
                  _full(W2.shape), _full((1, nd))],
        out_specs=_rows(r, nd),
        out_shape=jax.ShapeDtypeStruct((n, nd), F32),
    )(h, W1, b1.reshape(1, -1), W2, b2.reshape(1, -1))


@functools.lru_cache(maxsize=None)
def _make_gather_add(E, Hd):
    nw = NC * NS
    n_chunks = (E // nw) // CH
    mesh = plsc.VectorSubcoreMesh(core_axis_name="c", subcore_axis_name="s")

    K = 20

    @functools.partial(
        pl.kernel,
        out_type=jax.ShapeDtypeStruct((E, Hd), F32),
        mesh=mesh,
        scratch_types=[
            pltpu.VMEM((K, CH), jnp.int32),
            pltpu.VMEM((K, CH), jnp.int32),
            pltpu.VMEM((K, CH, Hd), F32),
            pltpu.SemaphoreType.DMA,
        ],
        compiler_params=pltpu.CompilerParams(use_tc_tiling_on_sc=False),
    )
    def gather_add(gs_hbm, gd_hbm, src_hbm, dst_hbm, out_hbm, src_v, dst_v,
                   rows_v, sem):
        wid = lax.axis_index("c") * NS + lax.axis_index("s")
        base = wid * n_chunks

        def do_chunks(j0, nk):
            ds = [pltpu.async_copy(src_hbm.at[wid, pl.ds(j0, nk)],
                                   src_v.at[pl.ds(0, nk)], sem),
                  pltpu.async_copy(dst_hbm.at[wid, pl.ds(j0, nk)],
                                   dst_v.at[pl.ds(0, nk)], sem)]
            for d in ds:
                d.wait()
            ds = [pltpu.async_copy(gs_hbm.at[src_v.at[b]], rows_v.at[b],
                                   sem) for b in range(nk)]
            for d in ds:
                d.wait()
            ds = [pltpu.async_copy(gd_hbm.at[dst_v.at[b]], rows_v.at[b],
                                   sem, add=True) for b in range(nk)]
            for d in ds:
                d.wait()
            ds = [pltpu.async_copy(
                rows_v.at[b], out_hbm.at[pl.ds((base + j0 + b) * CH, CH)],
                sem) for b in range(nk)]
            for d in ds:
                d.wait()

        def group(g, carry):
            do_chunks(g * K, K)
            return carry

        lax.fori_loop(0, n_chunks // K, group, 0)
        if n_chunks % K:
            do_chunks((n_chunks // K) * K, n_chunks % K)

    return gather_add


@functools.lru_cache(maxsize=None)
def _make_segment_sum(E, Hd):
    ec = Hd // 4
    n_chunks = (E // NS) // CH
    stripe = NP // NS
    mesh = plsc.VectorSubcoreMesh(core_axis_name="c", subcore_axis_name="s")

    K = 20

    @functools.partial(
        pl.kernel,
        out_type=jax.ShapeDtypeStruct((NP, Hd), F32),
        mesh=mesh,
        scratch_types=[
            pltpu.VMEM((K, CH), jnp.int32),
            pltpu.VMEM((K, CH, ec), F32),
            pltpu.VMEM_SHARED((NP, ec), F32),
            pltpu.SemaphoreType.DMA,
        ],
        compiler_params=pltpu.CompilerParams(use_tc_tiling_on_sc=False),
    )
    def seg_sum(e_hbm, dst_hbm, zeros_hbm, agg_hbm, dst_v, rows_c, slab, sem):
        core = lax.axis_index("c")
        sid = lax.axis_index("s")
        base = sid * n_chunks
        for p in range(2):
            pltpu.sync_copy(zeros_hbm, slab.at[pl.ds(sid * stripe, stripe)])
            plsc.subcore_barrier()
            col = ec * 2 * core + ec * p

            def do_chunks(j0, nk, col=col):
                ds = [pltpu.async_copy(dst_hbm.at[sid, pl.ds(j0, nk)],
                                       dst_v.at[pl.ds(0, nk)], sem)]
                ds += [pltpu.async_copy(
                    e_hbm.at[pl.ds((base + j0 + b) * CH, CH),
                             pl.ds(col, ec)], rows_c.at[b],
                    sem) for b in range(nk)]
                for d in ds:
                    d.wait()
                ds = [pltpu.async_copy(rows_c.at[b],
                                       slab.at[dst_v.at[b]],
                                       sem, add=True) for b in range(nk)]
                for d in ds:
                    d.wait()

            def group(g, carry):
                do_chunks(g * K, K)
                return carry

            lax.fori_loop(0, n_chunks // K, group, 0)
            if n_chunks % K:
                do_chunks((n_chunks // K) * K, n_chunks % K)
            plsc.subcore_barrier()
            pltpu.sync_copy(
                slab.at[pl.ds(sid * stripe, stripe)],
                agg_hbm.at[pl.ds(sid * stripe, stripe), pl.ds(col, ec)])

    return seg_sum


def kernel(pos, x, edge_index, edge_attr, params):
    n = pos.shape[0]
    e_cnt = edge_attr.shape[0]
    nw = NC * NS
    src3 = edge_index[0].astype(jnp.int32).reshape(nw, (e_cnt // nw) // CH, CH)
    dst3 = edge_index[1].astype(jnp.int32).reshape(nw, (e_cnt // nw) // CH, CH)
    dst16 = edge_index[1].astype(jnp.int32).reshape(NS, (e_cnt // NS) // CH, CH)
    zeros = jnp.zeros((NP // NS, 32), F32)

    (We1, be1), (We2, be2) = params['node_enc']
    h = _node_encoder(pos, x[:, 2:], params['B'], We1, be1, We2, be2,
                      *params['node_enc_ln'])
    (Wf1, bf1), (Wf2, bf2) = params['edge_enc']
    e = _edge_encoder(edge_attr, Wf1, bf1, Wf2, bf2, *params['edge_enc_ln'])

    gather_add = _make_gather_add(e_cnt, 128)
    seg_sum = _make_segment_sum(e_cnt, 128)

    for blk in params['blocks']:
        (Wm1, bm1), (Wm2, bm2) = blk['edge_mlp']
        gs, gd = _prep_tables(h, Wm1[128:256], Wm1[256:384])
        gath = gather_add(gs, gd, src3, dst3)
        e = _edge_mlp(e, gath, Wm1[0:128], bm1, Wm2, bm2, *blk['edge_ln'])
        agg = seg_sum(e, dst16, zeros)
        (Wn1, bn1), (Wn2, bn2) = blk['node_mlp']
        h = _node_mlp(h, agg, Wn1[0:128], Wn1[128:256], bn1, Wn2, bn2,
                      *blk['node_ln'])

    (Wd1, bd1), (Wd2, bd2) = params['dec']
    return _decoder(h, Wd1, bd1, Wd2, bd2)
